# baseline (speedup 1.0000x reference)
import functools

import jax
import jax.numpy as jnp
from jax import lax
from jax.experimental import pallas as pl
from jax.experimental.pallas import tpu as pltpu
from jax.experimental.pallas import tpu_sc as plsc

S, D, C, K = 2048, 768, 24576, 32
EBLK = 1024
NEB = C // EBLK
DBLK = 512
NDB = C // DBLK
CW = 128
NCHUNK = C // CW
NC, NS, L = 2, 16, 16
NW = NC * NS
RPW = S // NW


def _encode_body(a_ref, w_ref, b_ref, o_ref, m_ref):
    a = a_ref[...]
    w = w_ref[...]
    acc = lax.dot_general(a, w, (((1,), (1,)), ((), ())),
                          preferred_element_type=jnp.float32)
    acc = acc + b_ref[0:1, :]
    o_ref[...] = acc
    cols = [jnp.max(acc[:, j * CW:(j + 1) * CW], axis=-1, keepdims=True)
            for j in range(EBLK // CW)]
    pad = jnp.full((S, CW - EBLK // CW), -jnp.inf, jnp.float32)
    m_ref[...] = jnp.concatenate(cols + [pad], axis=-1)


def _encode(act2d, W_enc, b_enc2d):
    return pl.pallas_call(
        _encode_body,
        grid=(NEB,),
        in_specs=[
            pl.BlockSpec((S, D), lambda i: (0, 0)),
            pl.BlockSpec((EBLK, D), lambda i: (i, 0)),
            pl.BlockSpec((8, EBLK), lambda i: (0, i)),
        ],
        out_specs=[
            pl.BlockSpec((S, EBLK), lambda i: (0, i)),
            pl.BlockSpec((S, CW), lambda i: (0, i)),
        ],
        out_shape=[
            jax.ShapeDtypeStruct((S, C), jnp.float32),
            jax.ShapeDtypeStruct((S, NEB * CW), jnp.float32),
        ],
    )(act2d, W_enc, b_enc2d)


def _decode_body(p_ref, w_ref, t_ref, o_ref):
    i = pl.program_id(0)
    p = p_ref[...]
    t = t_ref[...]
    masked = jnp.where(p >= t, p, 0.0)
    acc = lax.dot_general(masked, w_ref[...], (((1,), (1,)), ((), ())),
                          preferred_element_type=jnp.float32)

    @pl.when(i == 0)
    def _():
        o_ref[...] = acc

    @pl.when(i != 0)
    def _():
        o_ref[...] += acc


def _decode(pre_act, W_emb, thr):
    return pl.pallas_call(
        _decode_body,
        grid=(NDB,),
        in_specs=[
            pl.BlockSpec((S, DBLK), lambda i: (0, i)),
            pl.BlockSpec((D, DBLK), lambda i: (0, i)),
            pl.BlockSpec((S, 1), lambda i: (0, 0)),
        ],
        out_specs=pl.BlockSpec((S, D), lambda i: (0, 0)),
        out_shape=jax.ShapeDtypeStruct((S, D), jnp.float32),
    )(pre_act, W_emb, thr)


import numpy as np

_MSB = np.int32(-2147483648)


def _iota16():
    return lax.iota(jnp.int32, L)


def _monotone(xi):
    return jnp.where(xi >= 0, xi | _MSB, ~xi)


def _inv_monotone(u):
    return jnp.where(u < 0, u ^ _MSB, ~u)


def _uge(a, b):
    return (a ^ _MSB) >= (b ^ _MSB)


def _digit(u, lvl):
    return lax.shift_right_logical(u, jnp.int32(28 - 4 * lvl)) & jnp.int32(15)


def _one_level(src, nv0, n, rank, thresh, hist, buf, lvl, compact):
    for j in range(16):
        hist[pl.ds(j * L, L)] = jnp.zeros((L,), jnp.int32)
    if lvl == 0:
        nv = nv0
    else:
        nv = lax.div(n + 15, jnp.int32(16))

    def hist_body(i, carry):
        u = src(i)
        digit = _digit(u, lvl)
        if lvl == 0:
            mask = None
        else:
            mask = (_iota16() + i * L) < n
        plsc.addupdate_scatter(hist, [_iota16() * 16 + digit],
                               jnp.ones((L,), jnp.int32), mask=mask)
        return carry

    lax.fori_loop(0, nv, hist_body, jnp.int32(0))
    htot = hist[pl.ds(0, L)]
    for j in range(1, 16):
        htot = htot + hist[pl.ds(j * L, L)]
    scum = plsc.cumsum(lax.rev(htot, (0,)))
    k = jnp.max(plsc.all_reduce_ffs(scum >= rank))
    b = 15 - k
    prev = jnp.sum(jnp.where(_iota16() == (k - 1), scum, 0))
    rank = rank - prev
    bvec = jnp.broadcast_to(b, (L,)).astype(jnp.int32)
    thresh = thresh | lax.shift_left(bvec, jnp.int32(28 - 4 * lvl))
    if not compact:
        return n, rank, thresh

    def comp_body(i, base):
        u = src(i)
        digit = _digit(u, lvl)
        keep = digit == bvec
        if lvl != 0:
            keep = keep & ((_iota16() + i * L) < n)
        pos = base + plsc.cumsum(keep.astype(jnp.int32)) - 1
        plsc.store_scatter(buf, [pos], u, mask=keep)
        return base + plsc.all_reduce_population_count(keep)

    base = lax.fori_loop(0, nv, comp_body, jnp.zeros((L,), jnp.int32))
    return jnp.max(base), rank, thresh


def _radix_select(load_fn, nv0, n0, rank0, hist, buf, nlevels):
    thresh = jnp.zeros((L,), jnp.int32)
    n = jnp.int32(n0)
    rank = jnp.int32(rank0)

    def buf_load(i):
        return buf[pl.ds(i * L, L)]

    for lvl in range(nlevels):
        compact = lvl < nlevels - 1
        if lvl == 0:
            n, rank, thresh = _one_level(load_fn, nv0, n, rank, thresh,
                                         hist, buf, 0, compact)
        else:
            def do(args, lvl=lvl, compact=compact):
                nn, rr, tt = args
                return _one_level(buf_load, None, nn, rr, tt, hist, buf,
                                  lvl, compact)

            n, rank, thresh = lax.cond(n > L, do, lambda a: a,
                                       (n, rank, thresh))

    def fin(args):
        nn, rr, tt = args
        u = buf_load(0)
        valid = _iota16() < nn
        srt = plsc.sort_key_val(u ^ _MSB, u, mask=valid, descending=True)
        tt = jnp.broadcast_to(
            jnp.sum(jnp.where(_iota16() == rr - 1, srt[1], 0)), (L,)
        ).astype(jnp.int32)
        return nn, rr, tt

    n, rank, thresh = lax.cond(n <= L, fin, lambda a: a, (n, rank, thresh))
    return thresh


def _sc_body(cmax_hbm, pre2d_hbm, out_hbm,
             cmax_v, cmd_v, hist_v, sel_v, cand_v, ping_v, out_v, sem):
    cid = lax.axis_index("c")
    sid = lax.axis_index("s")
    wid = sid * NC + cid
    low8 = _iota16() < 8

    def row_body(r, carry):
        row = wid * RPW + r
        pltpu.sync_copy(cmax_hbm.at[row], cmax_v)

        for jb in range(NEB):
            v = _monotone(cmax_v[pl.ds(jb * CW, L)])
            plsc.store_compressed(cmd_v.at[pl.ds(jb * 8, L)], v, mask=low8)

        def cm_load(i):
            return cmd_v[pl.ds(i * L, L)]

        t_u = _radix_select(cm_load, NCHUNK // L, NCHUNK, K, hist_v,
                            ping_v, 4)

        rowbase = row * NCHUNK
        fill = jnp.broadcast_to(rowbase, (L,)).astype(jnp.int32)
        for j in range(NCHUNK // L):
            sel_v[pl.ds(j * L, L)] = fill

        def sel_body(j, w):
            u = cmd_v[pl.ds(j * L, L)]
            keep = _uge(u, t_u)
            ids = rowbase + j * L + _iota16()
            plsc.store_compressed(sel_v.at[pl.ds(w, L)], ids, mask=keep)
            return w + jnp.max(plsc.all_reduce_population_count(keep))

        m = lax.fori_loop(0, NCHUNK // L, sel_body, jnp.int32(0))

        g = lax.div(m + 47, jnp.int32(48))

        def gat_body(j, carry):
            pltpu.async_copy(
                pre2d_hbm.at[sel_v.at[pl.ds(j * 48, 48)]],
                cand_v.at[pl.ds(j * 48, 48)], sem).wait()
            return carry

        lax.fori_loop(0, g, gat_body, jnp.int32(0))

        def cand_load(i):
            chunk = i >> 3
            off = (i & 7) * L
            return _monotone(cand_v[chunk, pl.ds(off, L)])

        v_u = _radix_select(cand_load, m * 8, m * CW, K, hist_v, ping_v, 8)
        thr_bits = _inv_monotone(v_u)
        plsc.store_scatter(out_v, [jnp.broadcast_to(r, (L,)).astype(jnp.int32)],
                           thr_bits, mask=_iota16() == 0)
        return carry

    lax.fori_loop(0, RPW, row_body, jnp.int32(0))
    pltpu.sync_copy(out_v, out_hbm.at[pl.ds(wid * RPW, RPW)])


@functools.partial(
    pl.kernel,
    out_type=jax.ShapeDtypeStruct((S,), jnp.int32),
    mesh=plsc.VectorSubcoreMesh(core_axis_name="c", subcore_axis_name="s"),
    compiler_params=pltpu.CompilerParams(needs_layout_passes=False),
    scratch_types=[
        pltpu.VMEM((NEB * CW,), jnp.int32),
        pltpu.VMEM((NCHUNK + L,), jnp.int32),
        pltpu.VMEM((256,), jnp.int32),
        pltpu.VMEM((NCHUNK,), jnp.int32),
        pltpu.VMEM((NCHUNK, CW), jnp.int32),
        pltpu.VMEM((C + L,), jnp.int32),
        pltpu.VMEM((RPW,), jnp.int32),
        pltpu.SemaphoreType.DMA,
    ],
)
def _sc_thresh(cmax_hbm, pre2d_hbm, out_hbm,
               cmax_v, cmd_v, hist_v, sel_v, cand_v, ping_v, out_v, sem):
    _sc_body(cmax_hbm, pre2d_hbm, out_hbm,
             cmax_v, cmd_v, hist_v, sel_v, cand_v, ping_v, out_v, sem)


def kernel(activations, W_enc, b_enc, W_emb):
    B = activations.shape[0]
    act2d = activations.reshape(B * S, D)
    b2d = jnp.broadcast_to(b_enc.reshape(1, C), (8, C))
    pre_act, cmax = _encode(act2d, W_enc, b2d)
    cmax_bits = lax.bitcast_convert_type(cmax, jnp.int32)
    pre_bits = lax.bitcast_convert_type(pre_act, jnp.int32).reshape(
        S * NCHUNK, CW)
    thr_bits = _sc_thresh(cmax_bits, pre_bits)
    thr = lax.bitcast_convert_type(thr_bits, jnp.float32)
    out = _decode(pre_act, W_emb, thr.reshape(S, 1))
    return out.reshape(B, S, D)


# speedup vs baseline: 1.0949x; 1.0949x over previous
import functools

import jax
import jax.numpy as jnp
from jax import lax
from jax.experimental import pallas as pl
from jax.experimental.pallas import tpu as pltpu
from jax.experimental.pallas import tpu_sc as plsc

S, D, C, K = 2048, 768, 24576, 32
EBLK = 1024
NEB = C // EBLK
DBLK = 512
NDB = C // DBLK
CW = 128
NCHUNK = C // CW
CV = CW // 16
PADW = 128
GB = 48
NC, NS, L = 2, 16, 16
NW = NC * NS
RPW = S // NW


def _encode_body(a_ref, w_ref, b_ref, o_ref, m_ref):
    a = a_ref[...]
    w = w_ref[...]
    acc = lax.dot_general(a, w, (((1,), (1,)), ((), ())),
                          preferred_element_type=jnp.float32)
    acc = acc + b_ref[0:1, :]
    o_ref[...] = acc
    cols = [jnp.max(acc[:, j * CW:(j + 1) * CW], axis=-1, keepdims=True)
            for j in range(EBLK // CW)]
    pad = jnp.full((S, PADW - EBLK // CW), -jnp.inf, jnp.float32)
    m_ref[...] = jnp.concatenate(cols + [pad], axis=-1)


def _encode(act2d, W_enc, b_enc2d):
    return pl.pallas_call(
        _encode_body,
        grid=(NEB,),
        in_specs=[
            pl.BlockSpec((S, D), lambda i: (0, 0)),
            pl.BlockSpec((EBLK, D), lambda i: (i, 0)),
            pl.BlockSpec((8, EBLK), lambda i: (0, i)),
        ],
        out_specs=[
            pl.BlockSpec((S, EBLK), lambda i: (0, i)),
            pl.BlockSpec((S, PADW), lambda i: (0, i)),
        ],
        out_shape=[
            jax.ShapeDtypeStruct((S, C), jnp.float32),
            jax.ShapeDtypeStruct((S, NEB * PADW), jnp.float32),
        ],
    )(act2d, W_enc, b_enc2d)


def _decode_body(p_ref, w_ref, t_ref, o_ref):
    i = pl.program_id(0)
    p = p_ref[...]
    t = t_ref[...]
    masked = jnp.where(p >= t, p, 0.0)
    acc = lax.dot_general(masked, w_ref[...], (((1,), (1,)), ((), ())),
                          preferred_element_type=jnp.float32)

    @pl.when(i == 0)
    def _():
        o_ref[...] = acc

    @pl.when(i != 0)
    def _():
        o_ref[...] += acc


def _decode(pre_act, W_emb, thr):
    return pl.pallas_call(
        _decode_body,
        grid=(NDB,),
        in_specs=[
            pl.BlockSpec((S, DBLK), lambda i: (0, i)),
            pl.BlockSpec((D, DBLK), lambda i: (0, i)),
            pl.BlockSpec((S, 1), lambda i: (0, 0)),
        ],
        out_specs=pl.BlockSpec((S, D), lambda i: (0, 0)),
        out_shape=jax.ShapeDtypeStruct((S, D), jnp.float32),
    )(pre_act, W_emb, thr)


import numpy as np

_MSB = np.int32(-2147483648)


def _iota16():
    return lax.iota(jnp.int32, L)


def _monotone(xi):
    return jnp.where(xi >= 0, xi | _MSB, ~xi)


def _inv_monotone(u):
    return jnp.where(u < 0, u ^ _MSB, ~u)


def _uge(a, b):
    return (a ^ _MSB) >= (b ^ _MSB)


def _digit(u, lvl):
    return lax.shift_right_logical(u, jnp.int32(28 - 4 * lvl)) & jnp.int32(15)


def _one_level(src, nv0, n, rank, thresh, hist, buf, lvl, compact):
    for j in range(16):
        hist[pl.ds(j * L, L)] = jnp.zeros((L,), jnp.int32)
    if lvl == 0:
        nv = nv0
    else:
        nv = lax.div(n + 15, jnp.int32(16))

    def hist_body(i, carry):
        u = src(i)
        digit = _digit(u, lvl)
        if lvl == 0:
            mask = None
        else:
            mask = (_iota16() + i * L) < n
        plsc.addupdate_scatter(hist, [_iota16() * 16 + digit],
                               jnp.ones((L,), jnp.int32), mask=mask)
        return carry

    lax.fori_loop(0, nv, hist_body, jnp.int32(0))
    htot = hist[pl.ds(0, L)]
    for j in range(1, 16):
        htot = htot + hist[pl.ds(j * L, L)]
    scum = plsc.cumsum(lax.rev(htot, (0,)))
    k = jnp.max(plsc.all_reduce_ffs(scum >= rank))
    b = 15 - k
    prev = jnp.sum(jnp.where(_iota16() == (k - 1), scum, 0))
    rank = rank - prev
    bvec = jnp.broadcast_to(b, (L,)).astype(jnp.int32)
    thresh = thresh | lax.shift_left(bvec, jnp.int32(28 - 4 * lvl))
    if not compact:
        return n, rank, thresh

    def comp_body(i, base):
        u = src(i)
        digit = _digit(u, lvl)
        keep = digit == bvec
        if lvl != 0:
            keep = keep & ((_iota16() + i * L) < n)
        pos = base + plsc.cumsum(keep.astype(jnp.int32)) - 1
        plsc.store_scatter(buf, [pos], u, mask=keep)
        return base + plsc.all_reduce_population_count(keep)

    base = lax.fori_loop(0, nv, comp_body, jnp.zeros((L,), jnp.int32))
    return jnp.max(base), rank, thresh


def _radix_select(load_fn, nv0, n0, rank0, hist, buf, nlevels):
    thresh = jnp.zeros((L,), jnp.int32)
    n = jnp.int32(n0)
    rank = jnp.int32(rank0)

    def buf_load(i):
        return buf[pl.ds(i * L, L)]

    for lvl in range(nlevels):
        compact = lvl < nlevels - 1
        if lvl == 0:
            n, rank, thresh = _one_level(load_fn, nv0, n, rank, thresh,
                                         hist, buf, 0, compact)
        else:
            def do(args, lvl=lvl, compact=compact):
                nn, rr, tt = args
                return _one_level(buf_load, None, nn, rr, tt, hist, buf,
                                  lvl, compact)

            n, rank, thresh = lax.cond(n > L, do, lambda a: a,
                                       (n, rank, thresh))

    def fin(args):
        nn, rr, tt = args
        u = buf_load(0)
        valid = _iota16() < nn
        srt = plsc.sort_key_val(u ^ _MSB, u, mask=valid, descending=True)
        tt = jnp.broadcast_to(
            jnp.sum(jnp.where(_iota16() == rr - 1, srt[1], 0)), (L,)
        ).astype(jnp.int32)
        return nn, rr, tt

    n, rank, thresh = lax.cond(n <= L, fin, lambda a: a, (n, rank, thresh))
    return thresh


def _sc_body(cmax_hbm, pre2d_hbm, out_hbm, cm_a, cm_b, cmd_v, hist_v,
             sel_a, sel_b, cand_a, cand_b, ping_v, out_v,
             semc_a, semc_b, semg_a, semg_b):
    cid = lax.axis_index("c")
    sid = lax.axis_index("s")
    wid = sid * NC + cid
    row0 = wid * RPW

    def start_cmax(r, cm_v, semc):
        pltpu.async_copy(cmax_hbm.at[row0 + r], cm_v, semc)

    def stage_ab(r, cm_v, sel_v, cand_v, semc, semg):
        pltpu.make_async_copy(cmax_hbm.at[0], cm_v, semc).wait()

        low8 = _iota16() < 8
        for jb in range(NEB):
            v = _monotone(cm_v[pl.ds(jb * PADW, L)])
            plsc.store_compressed(cmd_v.at[pl.ds(jb * 8, L)], v, mask=low8)

        def cm_load(i):
            return cmd_v[pl.ds(i * L, L)]

        t_u = _radix_select(cm_load, NCHUNK // L, NCHUNK, K, hist_v,
                            ping_v, 4)

        rowbase = (row0 + r) * NCHUNK
        fill = jnp.broadcast_to(rowbase, (L,)).astype(jnp.int32)
        for j in range(NCHUNK // L):
            sel_v[pl.ds(j * L, L)] = fill

        def sel_body(j, w):
            u = cmd_v[pl.ds(j * L, L)]
            keep = _uge(u, t_u)
            ids = rowbase + j * L + _iota16()
            plsc.store_compressed(sel_v.at[pl.ds(w, L)], ids, mask=keep)
            return w + jnp.max(plsc.all_reduce_population_count(keep))

        m = lax.fori_loop(0, NCHUNK // L, sel_body, jnp.int32(0))
        g = lax.div(m + (GB - 1), jnp.int32(GB))

        def gat_body(j, carry):
            pltpu.async_copy(
                pre2d_hbm.at[sel_v.at[pl.ds(j * GB, GB)]],
                cand_v.at[pl.ds(j * GB, GB)], semg)
            return carry

        lax.fori_loop(0, g, gat_body, jnp.int32(0))
        return m, g

    def stage_c(r, cand_v, semg, m, g):
        def drain(j, carry):
            pltpu.make_async_copy(pre2d_hbm.at[pl.ds(0, GB)],
                                  cand_v.at[pl.ds(j * GB, GB)], semg).wait()
            return carry

        lax.fori_loop(0, g, drain, jnp.int32(0))

        def cand_load(i):
            chunk = lax.shift_right_logical(i, 3)
            off = (i & (CV - 1)) * L
            return _monotone(cand_v[chunk, pl.ds(off, L)])

        v_u = _radix_select(cand_load, m * CV, m * CW, K, hist_v, ping_v, 8)
        thr_bits = _inv_monotone(v_u)
        plsc.store_scatter(out_v, [jnp.broadcast_to(r, (L,)).astype(jnp.int32)],
                           thr_bits, mask=_iota16() == 0)

    bufs = ((cm_a, sel_a, cand_a, semc_a, semg_a),
            (cm_b, sel_b, cand_b, semc_b, semg_b))
    start_cmax(0, cm_a, semc_a)

    def pair_body(q, gs):
        mg = list(gs)
        for par in (0, 1):
            cm_v, sel_v, cand_v, semc, semg = bufs[par]
            cmo_v, _, _, semco, _ = bufs[1 - par]
            r = q * 2 + par

            def run_ab(_):
                def pf(c):
                    start_cmax(r + 1, cmo_v, semco)
                    return c

                lax.cond(r + 1 < RPW, pf, lambda c: c, jnp.int32(0))
                return stage_ab(r, cm_v, sel_v, cand_v, semc, semg)

            mg[par] = lax.cond(r < RPW, run_ab,
                               lambda _: (jnp.int32(0), jnp.int32(0)),
                               jnp.int32(0))

            _, cand_o, semg_o, mg_o = (bufs[1 - par][2], bufs[1 - par][2],
                                       bufs[1 - par][4], mg[1 - par])

            def run_c(c):
                stage_c(r - 1, cand_o, semg_o, mg_o[0], mg_o[1])
                return c

            lax.cond((r >= 1) & (r <= RPW), run_c, lambda c: c, jnp.int32(0))
        return tuple(mg)

    zz = (jnp.int32(0), jnp.int32(0))
    lax.fori_loop(0, RPW // 2 + 1, pair_body, (zz, zz))
    pltpu.sync_copy(out_v, out_hbm.at[pl.ds(row0, RPW)])


@functools.partial(
    pl.kernel,
    out_type=jax.ShapeDtypeStruct((S,), jnp.int32),
    mesh=plsc.VectorSubcoreMesh(core_axis_name="c", subcore_axis_name="s"),
    compiler_params=pltpu.CompilerParams(needs_layout_passes=False),
    scratch_types=[
        pltpu.VMEM((NEB * PADW,), jnp.int32),
        pltpu.VMEM((NEB * PADW,), jnp.int32),
        pltpu.VMEM((NCHUNK + L,), jnp.int32),
        pltpu.VMEM((256,), jnp.int32),
        pltpu.VMEM((NCHUNK,), jnp.int32),
        pltpu.VMEM((NCHUNK,), jnp.int32),
        pltpu.VMEM((NCHUNK, CW), jnp.int32),
        pltpu.VMEM((NCHUNK, CW), jnp.int32),
        pltpu.VMEM((C + L,), jnp.int32),
        pltpu.VMEM((RPW,), jnp.int32),
        pltpu.SemaphoreType.DMA,
        pltpu.SemaphoreType.DMA,
        pltpu.SemaphoreType.DMA,
        pltpu.SemaphoreType.DMA,
    ],
)
def _sc_thresh(cmax_hbm, pre2d_hbm, out_hbm, cm_a, cm_b, cmd_v, hist_v,
               sel_a, sel_b, cand_a, cand_b, ping_v, out_v,
               semc_a, semc_b, semg_a, semg_b):
    _sc_body(cmax_hbm, pre2d_hbm, out_hbm, cm_a, cm_b, cmd_v, hist_v,
             sel_a, sel_b, cand_a, cand_b, ping_v, out_v,
             semc_a, semc_b, semg_a, semg_b)


def kernel(activations, W_enc, b_enc, W_emb):
    B = activations.shape[0]
    act2d = activations.reshape(B * S, D)
    b2d = jnp.broadcast_to(b_enc.reshape(1, C), (8, C))
    pre_act, cmax = _encode(act2d, W_enc, b2d)
    cmax_bits = lax.bitcast_convert_type(cmax, jnp.int32)
    pre_bits = lax.bitcast_convert_type(pre_act, jnp.int32).reshape(
        S * NCHUNK, CW)
    thr_bits = _sc_thresh(cmax_bits, pre_bits)
    thr = lax.bitcast_convert_type(thr_bits, jnp.float32)
    out = _decode(pre_act, W_emb, thr.reshape(S, 1))
    return out.reshape(B, S, D)


# speedup vs baseline: 1.2144x; 1.1092x over previous
import functools

import jax
import jax.numpy as jnp
from jax import lax
from jax.experimental import pallas as pl
from jax.experimental.pallas import tpu as pltpu
from jax.experimental.pallas import tpu_sc as plsc

S, D, C, K = 2048, 768, 24576, 32
EBLK = 1024
NEB = C // EBLK
DBLK = 512
NDB = C // DBLK
CW = 128
NCHUNK = C // CW
CV = CW // 16
PADW = 128
GB = 48
NC, NS, L = 2, 16, 16
NW = NC * NS
RPW = S // NW


import numpy as np

_MSB = np.int32(-2147483648)


def _encode_body(a_ref, w_ref, b_ref, o_ref, m_ref):
    a = a_ref[...]
    w = w_ref[...]
    acc = lax.dot_general(a, w, (((1,), (1,)), ((), ())),
                          preferred_element_type=jnp.float32)
    acc = acc + b_ref[0:1, :]
    xi = lax.bitcast_convert_type(acc, jnp.int32)
    o_ref[...] = jnp.where(xi >= 0, xi, ~xi ^ _MSB)
    cm = jnp.concatenate(
        [jnp.max(acc[:, j * CW:(j + 1) * CW], axis=-1, keepdims=True)
         for j in range(EBLK // CW)], axis=-1)
    ci = lax.bitcast_convert_type(cm, jnp.int32)
    cmk = jnp.where(ci >= 0, ci, ~ci ^ _MSB)
    pad = jnp.full((S, PADW - EBLK // CW), _MSB, jnp.int32)
    m_ref[...] = jnp.concatenate([cmk, pad], axis=-1)


def _encode(act2d, W_enc, b_enc2d):
    return pl.pallas_call(
        _encode_body,
        grid=(NEB,),
        in_specs=[
            pl.BlockSpec((S, D), lambda i: (0, 0)),
            pl.BlockSpec((EBLK, D), lambda i: (i, 0)),
            pl.BlockSpec((8, EBLK), lambda i: (0, i)),
        ],
        out_specs=[
            pl.BlockSpec((S, EBLK), lambda i: (0, i)),
            pl.BlockSpec((S, PADW), lambda i: (0, i)),
        ],
        out_shape=[
            jax.ShapeDtypeStruct((S, C), jnp.int32),
            jax.ShapeDtypeStruct((S, NEB * PADW), jnp.int32),
        ],
    )(act2d, W_enc, b_enc2d)


def _decode_body(p_ref, w_ref, t_ref, o_ref):
    i = pl.program_id(0)
    skey = p_ref[...]
    t = t_ref[...]
    xi = jnp.where(skey >= 0, skey, ~(skey ^ _MSB))
    p = lax.bitcast_convert_type(xi, jnp.float32)
    masked = jnp.where(skey >= t, p, 0.0)
    acc = lax.dot_general(masked, w_ref[...], (((1,), (1,)), ((), ())),
                          preferred_element_type=jnp.float32)

    @pl.when(i == 0)
    def _():
        o_ref[...] = acc

    @pl.when(i != 0)
    def _():
        o_ref[...] += acc


def _decode(pre_act, W_emb, thr):
    return pl.pallas_call(
        _decode_body,
        grid=(NDB,),
        in_specs=[
            pl.BlockSpec((S, DBLK), lambda i: (0, i)),
            pl.BlockSpec((D, DBLK), lambda i: (0, i)),
            pl.BlockSpec((S, 1), lambda i: (0, 0)),
        ],
        out_specs=pl.BlockSpec((S, D), lambda i: (0, 0)),
        out_shape=jax.ShapeDtypeStruct((S, D), jnp.float32),
    )(pre_act, W_emb, thr)


def _iota16():
    return lax.iota(jnp.int32, L)


def _digit(skey, lvl):
    d = lax.shift_right_logical(skey, jnp.int32(28 - 4 * lvl)) & jnp.int32(15)
    if lvl == 0:
        d = d ^ jnp.int32(8)
    return d


def _one_level(src, nv0, n, rank, thresh, hist, buf, lvl, compact):
    for j in range(16):
        hist[pl.ds(j * L, L)] = jnp.zeros((L,), jnp.int32)
    if lvl == 0:
        nv = nv0
    else:
        nv = lax.div(n + 15, jnp.int32(16))

    def hist_body(i, carry):
        u = src(i)
        digit = _digit(u, lvl)
        if lvl == 0:
            mask = None
        else:
            mask = (_iota16() + i * L) < n
        plsc.addupdate_scatter(hist, [_iota16() * 16 + digit],
                               jnp.ones((L,), jnp.int32), mask=mask)
        return carry

    lax.fori_loop(0, nv, hist_body, jnp.int32(0))
    htot = hist[pl.ds(0, L)]
    for j in range(1, 16):
        htot = htot + hist[pl.ds(j * L, L)]
    scum = plsc.cumsum(lax.rev(htot, (0,)))
    k = jnp.max(plsc.all_reduce_ffs(scum >= rank))
    b = 15 - k
    prev = jnp.sum(jnp.where(_iota16() == (k - 1), scum, 0))
    rank = rank - prev
    bvec = jnp.broadcast_to(b, (L,)).astype(jnp.int32)
    bkey = bvec ^ jnp.int32(8) if lvl == 0 else bvec
    thresh = thresh | lax.shift_left(bkey, jnp.int32(28 - 4 * lvl))
    if not compact:
        return n, rank, thresh

    def comp_body(i, base):
        u = src(i)
        digit = _digit(u, lvl)
        keep = digit == bvec
        if lvl != 0:
            keep = keep & ((_iota16() + i * L) < n)
        pos = base + plsc.cumsum(keep.astype(jnp.int32)) - 1
        plsc.store_scatter(buf, [pos], u, mask=keep)
        return base + plsc.all_reduce_population_count(keep)

    base = lax.fori_loop(0, nv, comp_body, jnp.zeros((L,), jnp.int32))
    return jnp.max(base), rank, thresh


def _radix_select(load_fn, nv0, n0, rank0, hist, buf, nlevels):
    thresh = jnp.zeros((L,), jnp.int32)
    n = jnp.int32(n0)
    rank = jnp.int32(rank0)

    def buf_load(i):
        return buf[pl.ds(i * L, L)]

    for lvl in range(nlevels):
        compact = lvl < nlevels - 1
        if lvl == 0:
            n, rank, thresh = _one_level(load_fn, nv0, n, rank, thresh,
                                         hist, buf, 0, compact)
        else:
            def do(args, lvl=lvl, compact=compact):
                nn, rr, tt = args
                return _one_level(buf_load, None, nn, rr, tt, hist, buf,
                                  lvl, compact)

            n, rank, thresh = lax.cond(n > L, do, lambda a: a,
                                       (n, rank, thresh))

    def fin(args):
        nn, rr, tt = args
        u = buf_load(0)
        valid = _iota16() < nn
        srt = plsc.sort_key_val(u, u, mask=valid, descending=True)
        tt = jnp.broadcast_to(
            jnp.sum(jnp.where(_iota16() == rr - 1, srt[1], 0)), (L,)
        ).astype(jnp.int32)
        return nn, rr, tt

    n, rank, thresh = lax.cond(n <= L, fin, lambda a: a, (n, rank, thresh))
    return thresh


def _sc_body(cmax_hbm, pre2d_hbm, out_hbm, cm_a, cm_b, cmd_v, hist_v,
             sel_a, sel_b, cand_a, cand_b, ping_v, out_v,
             semc_a, semc_b, semg_a, semg_b):
    cid = lax.axis_index("c")
    sid = lax.axis_index("s")
    wid = sid * NC + cid
    row0 = wid * RPW

    def start_cmax(r, cm_v, semc):
        pltpu.async_copy(cmax_hbm.at[row0 + r], cm_v, semc)

    def stage_ab(r, cm_v, sel_v, cand_v, semc, semg):
        pltpu.make_async_copy(cmax_hbm.at[0], cm_v, semc).wait()

        low8 = _iota16() < 8
        for jb in range(NEB):
            v = cm_v[pl.ds(jb * PADW, L)]
            plsc.store_compressed(cmd_v.at[pl.ds(jb * 8, L)], v, mask=low8)

        def cm_load(i):
            return cmd_v[pl.ds(i * L, L)]

        t_u = _radix_select(cm_load, NCHUNK // L, NCHUNK, K, hist_v,
                            ping_v, 4)

        rowbase = (row0 + r) * NCHUNK
        fill = jnp.broadcast_to(rowbase, (L,)).astype(jnp.int32)
        for j in range(NCHUNK // L):
            sel_v[pl.ds(j * L, L)] = fill

        def sel_body(j, w):
            u = cmd_v[pl.ds(j * L, L)]
            keep = u >= t_u
            ids = rowbase + j * L + _iota16()
            plsc.store_compressed(sel_v.at[pl.ds(w, L)], ids, mask=keep)
            return w + jnp.max(plsc.all_reduce_population_count(keep))

        m = lax.fori_loop(0, NCHUNK // L, sel_body, jnp.int32(0))
        g = lax.div(m + (GB - 1), jnp.int32(GB))

        def gat_body(j, carry):
            pltpu.async_copy(
                pre2d_hbm.at[sel_v.at[pl.ds(j * GB, GB)]],
                cand_v.at[pl.ds(j * GB, GB)], semg)
            return carry

        lax.fori_loop(0, g, gat_body, jnp.int32(0))
        return m, g

    def stage_c(r, cand_v, semg, m, g):
        def drain(j, carry):
            pltpu.make_async_copy(pre2d_hbm.at[pl.ds(0, GB)],
                                  cand_v.at[pl.ds(j * GB, GB)], semg).wait()
            return carry

        lax.fori_loop(0, g, drain, jnp.int32(0))

        def cand_load(i):
            chunk = lax.shift_right_logical(i, 3)
            off = (i & (CV - 1)) * L
            return cand_v[chunk, pl.ds(off, L)]

        v_u = _radix_select(cand_load, m * CV, m * CW, K, hist_v, ping_v, 8)
        plsc.store_scatter(out_v, [jnp.broadcast_to(r, (L,)).astype(jnp.int32)],
                           v_u, mask=_iota16() == 0)

    bufs = ((cm_a, sel_a, cand_a, semc_a, semg_a),
            (cm_b, sel_b, cand_b, semc_b, semg_b))
    start_cmax(0, cm_a, semc_a)

    def pair_body(q, gs):
        mg = list(gs)
        for par in (0, 1):
            cm_v, sel_v, cand_v, semc, semg = bufs[par]
            cmo_v, _, _, semco, _ = bufs[1 - par]
            r = q * 2 + par

            def run_ab(_):
                def pf(c):
                    start_cmax(r + 1, cmo_v, semco)
                    return c

                lax.cond(r + 1 < RPW, pf, lambda c: c, jnp.int32(0))
                return stage_ab(r, cm_v, sel_v, cand_v, semc, semg)

            mg[par] = lax.cond(r < RPW, run_ab,
                               lambda _: (jnp.int32(0), jnp.int32(0)),
                               jnp.int32(0))

            _, cand_o, semg_o, mg_o = (bufs[1 - par][2], bufs[1 - par][2],
                                       bufs[1 - par][4], mg[1 - par])

            def run_c(c):
                stage_c(r - 1, cand_o, semg_o, mg_o[0], mg_o[1])
                return c

            lax.cond((r >= 1) & (r <= RPW), run_c, lambda c: c, jnp.int32(0))
        return tuple(mg)

    zz = (jnp.int32(0), jnp.int32(0))
    lax.fori_loop(0, RPW // 2 + 1, pair_body, (zz, zz))
    pltpu.sync_copy(out_v, out_hbm.at[pl.ds(row0, RPW)])


@functools.partial(
    pl.kernel,
    out_type=jax.ShapeDtypeStruct((S,), jnp.int32),
    mesh=plsc.VectorSubcoreMesh(core_axis_name="c", subcore_axis_name="s"),
    compiler_params=pltpu.CompilerParams(needs_layout_passes=False),
    scratch_types=[
        pltpu.VMEM((NEB * PADW,), jnp.int32),
        pltpu.VMEM((NEB * PADW,), jnp.int32),
        pltpu.VMEM((NCHUNK + L,), jnp.int32),
        pltpu.VMEM((256,), jnp.int32),
        pltpu.VMEM((NCHUNK,), jnp.int32),
        pltpu.VMEM((NCHUNK,), jnp.int32),
        pltpu.VMEM((NCHUNK, CW), jnp.int32),
        pltpu.VMEM((NCHUNK, CW), jnp.int32),
        pltpu.VMEM((C + L,), jnp.int32),
        pltpu.VMEM((RPW,), jnp.int32),
        pltpu.SemaphoreType.DMA,
        pltpu.SemaphoreType.DMA,
        pltpu.SemaphoreType.DMA,
        pltpu.SemaphoreType.DMA,
    ],
)
def _sc_thresh(cmax_hbm, pre2d_hbm, out_hbm, cm_a, cm_b, cmd_v, hist_v,
               sel_a, sel_b, cand_a, cand_b, ping_v, out_v,
               semc_a, semc_b, semg_a, semg_b):
    _sc_body(cmax_hbm, pre2d_hbm, out_hbm, cm_a, cm_b, cmd_v, hist_v,
             sel_a, sel_b, cand_a, cand_b, ping_v, out_v,
             semc_a, semc_b, semg_a, semg_b)


def kernel(activations, W_enc, b_enc, W_emb):
    B = activations.shape[0]
    act2d = activations.reshape(B * S, D)
    b2d = jnp.broadcast_to(b_enc.reshape(1, C), (8, C))
    pre_keys, cmax_keys = _encode(act2d, W_enc, b2d)
    thr_keys = _sc_thresh(cmax_keys, pre_keys.reshape(S * NCHUNK, CW))
    out = _decode(pre_keys, W_emb, thr_keys.reshape(S, 1))
    return out.reshape(B, S, D)


# speedup vs baseline: 1.2202x; 1.0048x over previous
import functools

import jax
import jax.numpy as jnp
from jax import lax
from jax.experimental import pallas as pl
from jax.experimental.pallas import tpu as pltpu
from jax.experimental.pallas import tpu_sc as plsc

S, D, C, K = 2048, 768, 24576, 32
EBLK = 1024
NEB = C // EBLK
DBLK = 512
NDB = C // DBLK
CW = 128
NCHUNK = C // CW
CV = CW // 16
PADW = 128
GB = 48
NC, NS, L = 2, 16, 16
NW = NC * NS
RPW = S // NW


import numpy as np

_MSB = np.int32(-2147483648)


def _encode_body(a_ref, w_ref, b_ref, o_ref, m_ref):
    a = a_ref[...]
    w = w_ref[...]
    acc = lax.dot_general(a, w, (((1,), (1,)), ((), ())),
                          preferred_element_type=jnp.float32)
    acc = acc + b_ref[0:1, :]
    xi = lax.bitcast_convert_type(acc, jnp.int32)
    o_ref[...] = jnp.where(xi >= 0, xi, ~xi ^ _MSB)
    cm = jnp.concatenate(
        [jnp.max(acc[:, j * CW:(j + 1) * CW], axis=-1, keepdims=True)
         for j in range(EBLK // CW)], axis=-1)
    ci = lax.bitcast_convert_type(cm, jnp.int32)
    cmk = jnp.where(ci >= 0, ci, ~ci ^ _MSB)
    pad = jnp.full((S, PADW - EBLK // CW), _MSB, jnp.int32)
    m_ref[...] = jnp.concatenate([cmk, pad], axis=-1)


def _encode(act2d, W_enc, b_enc2d):
    return pl.pallas_call(
        _encode_body,
        grid=(NEB,),
        in_specs=[
            pl.BlockSpec((S, D), lambda i: (0, 0)),
            pl.BlockSpec((EBLK, D), lambda i: (i, 0)),
            pl.BlockSpec((8, EBLK), lambda i: (0, i)),
        ],
        out_specs=[
            pl.BlockSpec((S, EBLK), lambda i: (0, i)),
            pl.BlockSpec((S, PADW), lambda i: (0, i)),
        ],
        out_shape=[
            jax.ShapeDtypeStruct((S, C), jnp.int32),
            jax.ShapeDtypeStruct((S, NEB * PADW), jnp.int32),
        ],
    )(act2d, W_enc, b_enc2d)


def _decode_body(p_ref, w_ref, t_ref, o_ref):
    i = pl.program_id(0)
    skey = p_ref[...]
    t = t_ref[...]
    xi = jnp.where(skey >= 0, skey, ~(skey ^ _MSB))
    p = lax.bitcast_convert_type(xi, jnp.float32)
    masked = jnp.where(skey >= t, p, 0.0)
    acc = lax.dot_general(masked, w_ref[...], (((1,), (1,)), ((), ())),
                          preferred_element_type=jnp.float32)

    @pl.when(i == 0)
    def _():
        o_ref[...] = acc

    @pl.when(i != 0)
    def _():
        o_ref[...] += acc


def _decode(pre_act, W_emb, thr):
    return pl.pallas_call(
        _decode_body,
        grid=(NDB,),
        in_specs=[
            pl.BlockSpec((S, DBLK), lambda i: (0, i)),
            pl.BlockSpec((D, DBLK), lambda i: (0, i)),
            pl.BlockSpec((S, 1), lambda i: (0, 0)),
        ],
        out_specs=pl.BlockSpec((S, D), lambda i: (0, 0)),
        out_shape=jax.ShapeDtypeStruct((S, D), jnp.float32),
    )(pre_act, W_emb, thr)


def _iota16():
    return lax.iota(jnp.int32, L)


def _digit(skey, lvl):
    d = lax.shift_right_logical(skey, jnp.int32(28 - 4 * lvl)) & jnp.int32(15)
    if lvl == 0:
        d = d ^ jnp.int32(8)
    return d


def _pick_bin(hist, rank, lvl, thresh):
    htot = hist[pl.ds(0, L)]
    for j in range(1, 16):
        htot = htot + hist[pl.ds(j * L, L)]
    scum = plsc.cumsum(lax.rev(htot, (0,)))
    k = jnp.max(plsc.all_reduce_ffs(scum >= rank))
    b = 15 - k
    prev = jnp.sum(jnp.where(_iota16() == (k - 1), scum, 0))
    rank = rank - prev
    bvec = jnp.broadcast_to(b, (L,)).astype(jnp.int32)
    bkey = bvec ^ jnp.int32(8) if lvl == 0 else bvec
    thresh = thresh | lax.shift_left(bkey, jnp.int32(28 - 4 * lvl))
    return rank, bvec, thresh


def _zero_hist(hist):
    for j in range(16):
        hist[pl.ds(j * L, L)] = jnp.zeros((L,), jnp.int32)


def _one_level(src, nv0, n, rank, thresh, hist, buf, lvl, compact):
    _zero_hist(hist)
    if lvl == 0:
        nv = nv0
    else:
        nv = lax.div(n + 15, jnp.int32(16))

    def hist_body(i, carry):
        u = src(i)
        digit = _digit(u, lvl)
        if lvl == 0:
            mask = None
        else:
            mask = (_iota16() + i * L) < n
        plsc.addupdate_scatter(hist, [_iota16() * 16 + digit],
                               jnp.ones((L,), jnp.int32), mask=mask)
        return carry

    lax.fori_loop(0, nv, hist_body, jnp.int32(0))
    rank, bvec, thresh = _pick_bin(hist, rank, lvl, thresh)
    if not compact:
        return n, rank, thresh

    def comp_body(i, base):
        u = src(i)
        digit = _digit(u, lvl)
        keep = digit == bvec
        if lvl != 0:
            keep = keep & ((_iota16() + i * L) < n)
        pos = base + plsc.cumsum(keep.astype(jnp.int32)) - 1
        plsc.store_scatter(buf, [pos], u, mask=keep)
        return base + plsc.all_reduce_population_count(keep)

    base = lax.fori_loop(0, nv, comp_body, jnp.zeros((L,), jnp.int32))
    return jnp.max(base), rank, thresh


def _rest_levels(n, rank, thresh, hist, buf, start, nlevels):
    def buf_load(i):
        return buf[pl.ds(i * L, L)]

    for lvl in range(start, nlevels):
        compact = lvl < nlevels - 1

        def do(args, lvl=lvl, compact=compact):
            nn, rr, tt = args
            return _one_level(buf_load, None, nn, rr, tt, hist, buf,
                              lvl, compact)

        n, rank, thresh = lax.cond(n > L, do, lambda a: a, (n, rank, thresh))

    def fin(args):
        nn, rr, tt = args
        u = buf_load(0)
        valid = _iota16() < nn
        srt = plsc.sort_key_val(u, u, mask=valid, descending=True)
        tt = jnp.broadcast_to(
            jnp.sum(jnp.where(_iota16() == rr - 1, srt[1], 0)), (L,)
        ).astype(jnp.int32)
        return nn, rr, tt

    n, rank, thresh = lax.cond(n <= L, fin, lambda a: a, (n, rank, thresh))
    return thresh


def _radix_select(load_fn, nv0, n0, rank0, hist, buf, nlevels):
    n, rank, thresh = _one_level(load_fn, nv0, jnp.int32(n0),
                                 jnp.int32(rank0), jnp.zeros((L,), jnp.int32),
                                 hist, buf, 0, True)
    return _rest_levels(n, rank, thresh, hist, buf, 1, nlevels)


def _cand_select(cand_v, m, rank0, hist, buf):
    _zero_hist(hist)
    ones = jnp.ones((L,), jnp.int32)

    def hist_body(c, carry):
        for k in range(CV):
            u = cand_v[c, pl.ds(k * L, L)]
            plsc.addupdate_scatter(hist, [_iota16() * 16 + _digit(u, 0)],
                                   ones, mask=None)
        return carry

    lax.fori_loop(0, m, hist_body, jnp.int32(0))
    rank, bvec, thresh = _pick_bin(hist, jnp.int32(rank0), 0,
                                   jnp.zeros((L,), jnp.int32))

    def comp_body(c, base):
        for k in range(CV):
            u = cand_v[c, pl.ds(k * L, L)]
            keep = _digit(u, 0) == bvec
            pos = base + plsc.cumsum(keep.astype(jnp.int32)) - 1
            plsc.store_scatter(buf, [pos], u, mask=keep)
            base = base + plsc.all_reduce_population_count(keep)
        return base

    base = lax.fori_loop(0, m, comp_body, jnp.zeros((L,), jnp.int32))
    return _rest_levels(jnp.max(base), rank, thresh, hist, buf, 1, 8)


def _sc_body(cmax_hbm, pre2d_hbm, out_hbm, cm_a, cm_b, cmd_v, hist_v,
             sel_a, sel_b, cand_a, cand_b, ping_v, out_v,
             semc_a, semc_b, semg_a, semg_b):
    cid = lax.axis_index("c")
    sid = lax.axis_index("s")
    wid = sid * NC + cid
    row0 = wid * RPW

    def start_cmax(r, cm_v, semc):
        pltpu.async_copy(cmax_hbm.at[row0 + r], cm_v, semc)

    def stage_ab(r, cm_v, sel_v, cand_v, semc, semg):
        pltpu.make_async_copy(cmax_hbm.at[0], cm_v, semc).wait()

        low8 = _iota16() < 8
        for jb in range(NEB):
            v = cm_v[pl.ds(jb * PADW, L)]
            plsc.store_compressed(cmd_v.at[pl.ds(jb * 8, L)], v, mask=low8)

        def cm_load(i):
            return cmd_v[pl.ds(i * L, L)]

        t_u = _radix_select(cm_load, NCHUNK // L, NCHUNK, K, hist_v,
                            ping_v, 4)

        rowbase = (row0 + r) * NCHUNK
        fill = jnp.broadcast_to(rowbase, (L,)).astype(jnp.int32)
        for j in range(NCHUNK // L):
            sel_v[pl.ds(j * L, L)] = fill

        def sel_body(j, w):
            u = cmd_v[pl.ds(j * L, L)]
            keep = u >= t_u
            ids = rowbase + j * L + _iota16()
            plsc.store_compressed(sel_v.at[pl.ds(w, L)], ids, mask=keep)
            return w + jnp.max(plsc.all_reduce_population_count(keep))

        m = lax.fori_loop(0, NCHUNK // L, sel_body, jnp.int32(0))
        g = lax.div(m + (GB - 1), jnp.int32(GB))

        def gat_body(j, carry):
            pltpu.async_copy(
                pre2d_hbm.at[sel_v.at[pl.ds(j * GB, GB)]],
                cand_v.at[pl.ds(j * GB, GB)], semg)
            return carry

        lax.fori_loop(0, g, gat_body, jnp.int32(0))
        return m, g

    def stage_c(r, cand_v, semg, m, g):
        def drain(j, carry):
            pltpu.make_async_copy(pre2d_hbm.at[pl.ds(0, GB)],
                                  cand_v.at[pl.ds(j * GB, GB)], semg).wait()
            return carry

        lax.fori_loop(0, g, drain, jnp.int32(0))

        def cand_load(i):
            chunk = lax.shift_right_logical(i, 3)
            off = (i & (CV - 1)) * L
            return cand_v[chunk, pl.ds(off, L)]

        v_u = _cand_select(cand_v, m, K, hist_v, ping_v)
        plsc.store_scatter(out_v, [jnp.broadcast_to(r, (L,)).astype(jnp.int32)],
                           v_u, mask=_iota16() == 0)

    bufs = ((cm_a, sel_a, cand_a, semc_a, semg_a),
            (cm_b, sel_b, cand_b, semc_b, semg_b))
    start_cmax(0, cm_a, semc_a)

    def pair_body(q, gs):
        mg = list(gs)
        for par in (0, 1):
            cm_v, sel_v, cand_v, semc, semg = bufs[par]
            cmo_v, _, _, semco, _ = bufs[1 - par]
            r = q * 2 + par

            def run_ab(_):
                def pf(c):
                    start_cmax(r + 1, cmo_v, semco)
                    return c

                lax.cond(r + 1 < RPW, pf, lambda c: c, jnp.int32(0))
                return stage_ab(r, cm_v, sel_v, cand_v, semc, semg)

            mg[par] = lax.cond(r < RPW, run_ab,
                               lambda _: (jnp.int32(0), jnp.int32(0)),
                               jnp.int32(0))

            _, cand_o, semg_o, mg_o = (bufs[1 - par][2], bufs[1 - par][2],
                                       bufs[1 - par][4], mg[1 - par])

            def run_c(c):
                stage_c(r - 1, cand_o, semg_o, mg_o[0], mg_o[1])
                return c

            lax.cond((r >= 1) & (r <= RPW), run_c, lambda c: c, jnp.int32(0))
        return tuple(mg)

    zz = (jnp.int32(0), jnp.int32(0))
    lax.fori_loop(0, RPW // 2 + 1, pair_body, (zz, zz))
    pltpu.sync_copy(out_v, out_hbm.at[pl.ds(row0, RPW)])


@functools.partial(
    pl.kernel,
    out_type=jax.ShapeDtypeStruct((S,), jnp.int32),
    mesh=plsc.VectorSubcoreMesh(core_axis_name="c", subcore_axis_name="s"),
    compiler_params=pltpu.CompilerParams(needs_layout_passes=False),
    scratch_types=[
        pltpu.VMEM((NEB * PADW,), jnp.int32),
        pltpu.VMEM((NEB * PADW,), jnp.int32),
        pltpu.VMEM((NCHUNK + L,), jnp.int32),
        pltpu.VMEM((256,), jnp.int32),
        pltpu.VMEM((NCHUNK,), jnp.int32),
        pltpu.VMEM((NCHUNK,), jnp.int32),
        pltpu.VMEM((NCHUNK, CW), jnp.int32),
        pltpu.VMEM((NCHUNK, CW), jnp.int32),
        pltpu.VMEM((C + L,), jnp.int32),
        pltpu.VMEM((RPW,), jnp.int32),
        pltpu.SemaphoreType.DMA,
        pltpu.SemaphoreType.DMA,
        pltpu.SemaphoreType.DMA,
        pltpu.SemaphoreType.DMA,
    ],
)
def _sc_thresh(cmax_hbm, pre2d_hbm, out_hbm, cm_a, cm_b, cmd_v, hist_v,
               sel_a, sel_b, cand_a, cand_b, ping_v, out_v,
               semc_a, semc_b, semg_a, semg_b):
    _sc_body(cmax_hbm, pre2d_hbm, out_hbm, cm_a, cm_b, cmd_v, hist_v,
             sel_a, sel_b, cand_a, cand_b, ping_v, out_v,
             semc_a, semc_b, semg_a, semg_b)


def kernel(activations, W_enc, b_enc, W_emb):
    B = activations.shape[0]
    act2d = activations.reshape(B * S, D)
    b2d = jnp.broadcast_to(b_enc.reshape(1, C), (8, C))
    pre_keys, cmax_keys = _encode(act2d, W_enc, b2d)
    thr_keys = _sc_thresh(cmax_keys, pre_keys.reshape(S * NCHUNK, CW))
    out = _decode(pre_keys, W_emb, thr_keys.reshape(S, 1))
    return out.reshape(B, S, D)


# speedup vs baseline: 1.8163x; 1.4885x over previous
import functools

import jax
import jax.numpy as jnp
from jax import lax
from jax.experimental import pallas as pl
from jax.experimental.pallas import tpu as pltpu
from jax.experimental.pallas import tpu_sc as plsc

S, D, C, K = 2048, 768, 24576, 32
EBLK = 1024
NEB = C // EBLK
DBLK = 512
NDB = C // DBLK
CW = 128
NCHUNK = C // CW
CV = CW // 16
PADW = 128
GB = 48
NC, NS, L = 2, 16, 16
NW = NC * NS
RPW = S // NW


import numpy as np

_MSB = np.int32(-2147483648)


def _encode_body(a_ref, w_ref, b_ref, o_ref, m_ref):
    a = a_ref[...]
    w = w_ref[...]
    acc = lax.dot_general(a, w, (((1,), (1,)), ((), ())),
                          preferred_element_type=jnp.float32)
    acc = acc + b_ref[0:1, :]
    xi = lax.bitcast_convert_type(acc, jnp.int32)
    o_ref[...] = jnp.where(xi >= 0, xi, ~xi ^ _MSB)
    cm = jnp.concatenate(
        [jnp.max(acc[:, j * CW:(j + 1) * CW], axis=-1, keepdims=True)
         for j in range(EBLK // CW)], axis=-1)
    ci = lax.bitcast_convert_type(cm, jnp.int32)
    cmk = jnp.where(ci >= 0, ci, ~ci ^ _MSB)
    pad = jnp.full((S, PADW - EBLK // CW), _MSB, jnp.int32)
    m_ref[...] = jnp.concatenate([cmk, pad], axis=-1)


def _encode(act2d, W_enc, b_enc2d):
    return pl.pallas_call(
        _encode_body,
        grid=(NEB,),
        in_specs=[
            pl.BlockSpec((S, D), lambda i: (0, 0)),
            pl.BlockSpec((EBLK, D), lambda i: (i, 0)),
            pl.BlockSpec((8, EBLK), lambda i: (0, i)),
        ],
        out_specs=[
            pl.BlockSpec((S, EBLK), lambda i: (0, i)),
            pl.BlockSpec((S, PADW), lambda i: (0, i)),
        ],
        out_shape=[
            jax.ShapeDtypeStruct((S, C), jnp.int32),
            jax.ShapeDtypeStruct((S, NEB * PADW), jnp.int32),
        ],
    )(act2d, W_enc, b_enc2d)


def _decode_body(p_ref, w_ref, t_ref, o_ref):
    i = pl.program_id(0)
    skey = p_ref[...]
    t = t_ref[...]
    xi = jnp.where(skey >= 0, skey, ~(skey ^ _MSB))
    p = lax.bitcast_convert_type(xi, jnp.float32)
    masked = jnp.where(skey >= t, p, 0.0)
    acc = lax.dot_general(masked, w_ref[...], (((1,), (1,)), ((), ())),
                          preferred_element_type=jnp.float32)

    @pl.when(i == 0)
    def _():
        o_ref[...] = acc

    @pl.when(i != 0)
    def _():
        o_ref[...] += acc


def _decode(pre_act, W_emb, thr):
    return pl.pallas_call(
        _decode_body,
        grid=(NDB,),
        in_specs=[
            pl.BlockSpec((S, DBLK), lambda i: (0, i)),
            pl.BlockSpec((D, DBLK), lambda i: (0, i)),
            pl.BlockSpec((S, 1), lambda i: (0, 0)),
        ],
        out_specs=pl.BlockSpec((S, D), lambda i: (0, 0)),
        out_shape=jax.ShapeDtypeStruct((S, D), jnp.float32),
    )(pre_act, W_emb, thr)


def _iota16():
    return lax.iota(jnp.int32, L)


def _digit(skey, lvl):
    d = lax.shift_right_logical(skey, jnp.int32(28 - 4 * lvl)) & jnp.int32(15)
    if lvl == 0:
        d = d ^ jnp.int32(8)
    return d


def _pick_bin(hist, rank, lvl, thresh):
    htot = hist[pl.ds(0, L)]
    for j in range(1, 16):
        htot = htot + hist[pl.ds(j * L, L)]
    scum = plsc.cumsum(lax.rev(htot, (0,)))
    k = jnp.max(plsc.all_reduce_ffs(scum >= rank))
    b = 15 - k
    prev = jnp.sum(jnp.where(_iota16() == (k - 1), scum, 0))
    rank = rank - prev
    bvec = jnp.broadcast_to(b, (L,)).astype(jnp.int32)
    bkey = bvec ^ jnp.int32(8) if lvl == 0 else bvec
    thresh = thresh | lax.shift_left(bkey, jnp.int32(28 - 4 * lvl))
    return rank, bvec, thresh


def _zero_hist(hist):
    for j in range(16):
        hist[pl.ds(j * L, L)] = jnp.zeros((L,), jnp.int32)


def _one_level(src, nv0, n, rank, thresh, hist, buf, lvl, compact):
    _zero_hist(hist)
    if lvl == 0:
        nv = nv0
    else:
        nv = lax.div(n + 15, jnp.int32(16))

    def hist_body(i, carry):
        u = src(i)
        digit = _digit(u, lvl)
        if lvl == 0:
            mask = None
        else:
            mask = (_iota16() + i * L) < n
        plsc.addupdate_scatter(hist, [_iota16() * 16 + digit],
                               jnp.ones((L,), jnp.int32), mask=mask)
        return carry

    lax.fori_loop(0, nv, hist_body, jnp.int32(0))
    rank, bvec, thresh = _pick_bin(hist, rank, lvl, thresh)
    if not compact:
        return n, rank, thresh

    def comp_body(i, base):
        u = src(i)
        digit = _digit(u, lvl)
        keep = digit == bvec
        if lvl != 0:
            keep = keep & ((_iota16() + i * L) < n)
        pos = base + plsc.cumsum(keep.astype(jnp.int32)) - 1
        plsc.store_scatter(buf, [pos], u, mask=keep)
        return base + plsc.all_reduce_population_count(keep)

    base = lax.fori_loop(0, nv, comp_body, jnp.zeros((L,), jnp.int32))
    return jnp.max(base), rank, thresh


def _rest_levels(n, rank, thresh, hist, buf, start, nlevels):
    def buf_load(i):
        return buf[pl.ds(i * L, L)]

    for lvl in range(start, nlevels):
        compact = lvl < nlevels - 1

        def do(args, lvl=lvl, compact=compact):
            nn, rr, tt = args
            return _one_level(buf_load, None, nn, rr, tt, hist, buf,
                              lvl, compact)

        n, rank, thresh = lax.cond(n > L, do, lambda a: a, (n, rank, thresh))

    def fin(args):
        nn, rr, tt = args
        u = buf_load(0)
        valid = _iota16() < nn
        srt = plsc.sort_key_val(u, u, mask=valid, descending=True)
        tt = jnp.broadcast_to(
            jnp.sum(jnp.where(_iota16() == rr - 1, srt[1], 0)), (L,)
        ).astype(jnp.int32)
        return nn, rr, tt

    n, rank, thresh = lax.cond(n <= L, fin, lambda a: a, (n, rank, thresh))
    return thresh


def _radix_select(load_fn, nv0, n0, rank0, hist, buf, nlevels):
    n, rank, thresh = _one_level(load_fn, nv0, jnp.int32(n0),
                                 jnp.int32(rank0), jnp.zeros((L,), jnp.int32),
                                 hist, buf, 0, True)
    return _rest_levels(n, rank, thresh, hist, buf, 1, nlevels)


def _cand_select(cand_v, m, t_u, rank0, hist, buf):
    def comp_body(c, base):
        for k in range(CV):
            u = cand_v[c, pl.ds(k * L, L)]
            keep = u >= t_u
            pos = base + plsc.cumsum(keep.astype(jnp.int32)) - 1
            plsc.store_scatter(buf, [pos], u, mask=keep)
            base = base + plsc.all_reduce_population_count(keep)
        return base

    base = lax.fori_loop(0, m, comp_body, jnp.zeros((L,), jnp.int32))
    n1 = jnp.max(base)
    plsc.store_scatter(buf, [n1 + _iota16()],
                       jnp.full((L,), _MSB, jnp.int32), mask=None)

    def buf_load(i):
        return buf[pl.ds(i * L, L)]

    nv = lax.div(n1 + 15, jnp.int32(16))
    n, rank, thresh = _one_level(buf_load, nv, nv * L, jnp.int32(rank0),
                                 jnp.zeros((L,), jnp.int32), hist, buf,
                                 0, True)
    return _rest_levels(n, rank, thresh, hist, buf, 1, 8)


def _sc_body(cmax_hbm, pre2d_hbm, out_hbm, cm_a, cm_b, cmd_v, hist_v,
             sel_a, sel_b, cand_a, cand_b, ping_v, out_v,
             semc_a, semc_b, semg_a, semg_b):
    cid = lax.axis_index("c")
    sid = lax.axis_index("s")
    wid = sid * NC + cid
    row0 = wid * RPW

    def start_cmax(r, cm_v, semc):
        pltpu.async_copy(cmax_hbm.at[row0 + r], cm_v, semc)

    def stage_ab(r, cm_v, sel_v, cand_v, semc, semg):
        pltpu.make_async_copy(cmax_hbm.at[0], cm_v, semc).wait()

        low8 = _iota16() < 8
        for jb in range(NEB):
            v = cm_v[pl.ds(jb * PADW, L)]
            plsc.store_compressed(cmd_v.at[pl.ds(jb * 8, L)], v, mask=low8)

        def cm_load(i):
            return cmd_v[pl.ds(i * L, L)]

        t_u = _radix_select(cm_load, NCHUNK // L, NCHUNK, K, hist_v,
                            ping_v, 4)

        rowbase = (row0 + r) * NCHUNK
        fill = jnp.broadcast_to(rowbase, (L,)).astype(jnp.int32)
        for j in range(NCHUNK // L):
            sel_v[pl.ds(j * L, L)] = fill

        def sel_body(j, w):
            u = cmd_v[pl.ds(j * L, L)]
            keep = u >= t_u
            ids = rowbase + j * L + _iota16()
            plsc.store_compressed(sel_v.at[pl.ds(w, L)], ids, mask=keep)
            return w + jnp.max(plsc.all_reduce_population_count(keep))

        m = lax.fori_loop(0, NCHUNK // L, sel_body, jnp.int32(0))
        g = lax.div(m + (GB - 1), jnp.int32(GB))

        def gat_body(j, carry):
            pltpu.async_copy(
                pre2d_hbm.at[sel_v.at[pl.ds(j * GB, GB)]],
                cand_v.at[pl.ds(j * GB, GB)], semg)
            return carry

        lax.fori_loop(0, g, gat_body, jnp.int32(0))
        return m, g, t_u

    def stage_c(r, cand_v, semg, m, g, t_u):
        def drain(j, carry):
            pltpu.make_async_copy(pre2d_hbm.at[pl.ds(0, GB)],
                                  cand_v.at[pl.ds(j * GB, GB)], semg).wait()
            return carry

        lax.fori_loop(0, g, drain, jnp.int32(0))

        def cand_load(i):
            chunk = lax.shift_right_logical(i, 3)
            off = (i & (CV - 1)) * L
            return cand_v[chunk, pl.ds(off, L)]

        v_u = _cand_select(cand_v, m, t_u, K, hist_v, ping_v)
        plsc.store_scatter(out_v, [jnp.broadcast_to(r, (L,)).astype(jnp.int32)],
                           v_u, mask=_iota16() == 0)

    bufs = ((cm_a, sel_a, cand_a, semc_a, semg_a),
            (cm_b, sel_b, cand_b, semc_b, semg_b))
    start_cmax(0, cm_a, semc_a)

    def pair_body(q, gs):
        mg = list(gs)
        for par in (0, 1):
            cm_v, sel_v, cand_v, semc, semg = bufs[par]
            cmo_v, _, _, semco, _ = bufs[1 - par]
            r = q * 2 + par

            def run_ab(_):
                def pf(c):
                    start_cmax(r + 1, cmo_v, semco)
                    return c

                lax.cond(r + 1 < RPW, pf, lambda c: c, jnp.int32(0))
                return stage_ab(r, cm_v, sel_v, cand_v, semc, semg)

            mg[par] = lax.cond(r < RPW, run_ab,
                               lambda _: (jnp.int32(0), jnp.int32(0),
                                          jnp.zeros((L,), jnp.int32)),
                               jnp.int32(0))

            _, cand_o, semg_o, mg_o = (bufs[1 - par][2], bufs[1 - par][2],
                                       bufs[1 - par][4], mg[1 - par])

            def run_c(c):
                stage_c(r - 1, cand_o, semg_o, mg_o[0], mg_o[1], mg_o[2])
                return c

            lax.cond((r >= 1) & (r <= RPW), run_c, lambda c: c, jnp.int32(0))
        return tuple(mg)

    zz = (jnp.int32(0), jnp.int32(0), jnp.zeros((L,), jnp.int32))
    lax.fori_loop(0, RPW // 2 + 1, pair_body, (zz, zz))
    pltpu.sync_copy(out_v, out_hbm.at[pl.ds(row0, RPW)])


@functools.partial(
    pl.kernel,
    out_type=jax.ShapeDtypeStruct((S,), jnp.int32),
    mesh=plsc.VectorSubcoreMesh(core_axis_name="c", subcore_axis_name="s"),
    compiler_params=pltpu.CompilerParams(needs_layout_passes=False),
    scratch_types=[
        pltpu.VMEM((NEB * PADW,), jnp.int32),
        pltpu.VMEM((NEB * PADW,), jnp.int32),
        pltpu.VMEM((NCHUNK + L,), jnp.int32),
        pltpu.VMEM((256,), jnp.int32),
        pltpu.VMEM((NCHUNK,), jnp.int32),
        pltpu.VMEM((NCHUNK,), jnp.int32),
        pltpu.VMEM((NCHUNK, CW), jnp.int32),
        pltpu.VMEM((NCHUNK, CW), jnp.int32),
        pltpu.VMEM((C + L,), jnp.int32),
        pltpu.VMEM((RPW,), jnp.int32),
        pltpu.SemaphoreType.DMA,
        pltpu.SemaphoreType.DMA,
        pltpu.SemaphoreType.DMA,
        pltpu.SemaphoreType.DMA,
    ],
)
def _sc_thresh(cmax_hbm, pre2d_hbm, out_hbm, cm_a, cm_b, cmd_v, hist_v,
               sel_a, sel_b, cand_a, cand_b, ping_v, out_v,
               semc_a, semc_b, semg_a, semg_b):
    _sc_body(cmax_hbm, pre2d_hbm, out_hbm, cm_a, cm_b, cmd_v, hist_v,
             sel_a, sel_b, cand_a, cand_b, ping_v, out_v,
             semc_a, semc_b, semg_a, semg_b)


def kernel(activations, W_enc, b_enc, W_emb):
    B = activations.shape[0]
    act2d = activations.reshape(B * S, D)
    b2d = jnp.broadcast_to(b_enc.reshape(1, C), (8, C))
    pre_keys, cmax_keys = _encode(act2d, W_enc, b2d)
    thr_keys = _sc_thresh(cmax_keys, pre_keys.reshape(S * NCHUNK, CW))
    out = _decode(pre_keys, W_emb, thr_keys.reshape(S, 1))
    return out.reshape(B, S, D)


# speedup vs baseline: 1.9559x; 1.0769x over previous
import functools

import jax
import jax.numpy as jnp
from jax import lax
from jax.experimental import pallas as pl
from jax.experimental.pallas import tpu as pltpu
from jax.experimental.pallas import tpu_sc as plsc

S, D, C, K = 2048, 768, 24576, 32
EBLK = 1024
NEB = C // EBLK
DBLK = 512
NDB = C // DBLK
CW = 128
NCHUNK = C // CW
CV = CW // 16
PADW = 128
GB = 48
NC, NS, L = 2, 16, 16
NW = NC * NS
RPW = S // NW


import numpy as np

_MSB = np.int32(-2147483648)


def _encode_body(a_ref, w_ref, b_ref, o_ref, m_ref):
    a = a_ref[...]
    w = w_ref[...]
    acc = lax.dot_general(a, w, (((1,), (1,)), ((), ())),
                          preferred_element_type=jnp.float32)
    acc = acc + b_ref[0:1, :]
    xi = lax.bitcast_convert_type(acc, jnp.int32)
    o_ref[...] = jnp.where(xi >= 0, xi, ~xi ^ _MSB)
    cm = jnp.concatenate(
        [jnp.max(acc[:, j * CW:(j + 1) * CW], axis=-1, keepdims=True)
         for j in range(EBLK // CW)], axis=-1)
    ci = lax.bitcast_convert_type(cm, jnp.int32)
    cmk = jnp.where(ci >= 0, ci, ~ci ^ _MSB)
    pad = jnp.full((S, PADW - EBLK // CW), _MSB, jnp.int32)
    m_ref[...] = jnp.concatenate([cmk, pad], axis=-1)


def _encode(act2d, W_enc, b_enc2d):
    return pl.pallas_call(
        _encode_body,
        grid=(NEB,),
        in_specs=[
            pl.BlockSpec((S, D), lambda i: (0, 0)),
            pl.BlockSpec((EBLK, D), lambda i: (i, 0)),
            pl.BlockSpec((8, EBLK), lambda i: (0, i)),
        ],
        out_specs=[
            pl.BlockSpec((S, EBLK), lambda i: (0, i)),
            pl.BlockSpec((S, PADW), lambda i: (0, i)),
        ],
        out_shape=[
            jax.ShapeDtypeStruct((S, C), jnp.int32),
            jax.ShapeDtypeStruct((S, NEB * PADW), jnp.int32),
        ],
    )(act2d, W_enc, b_enc2d)


def _decode_body(p_ref, w_ref, t_ref, o_ref):
    i = pl.program_id(0)
    skey = p_ref[...]
    t = t_ref[...]
    xi = jnp.where(skey >= 0, skey, ~(skey ^ _MSB))
    p = lax.bitcast_convert_type(xi, jnp.float32)
    masked = jnp.where(skey >= t, p, 0.0)
    acc = lax.dot_general(masked, w_ref[...], (((1,), (1,)), ((), ())),
                          preferred_element_type=jnp.float32)

    @pl.when(i == 0)
    def _():
        o_ref[...] = acc

    @pl.when(i != 0)
    def _():
        o_ref[...] += acc


def _decode(pre_act, W_emb, thr):
    return pl.pallas_call(
        _decode_body,
        grid=(NDB,),
        in_specs=[
            pl.BlockSpec((S, DBLK), lambda i: (0, i)),
            pl.BlockSpec((D, DBLK), lambda i: (0, i)),
            pl.BlockSpec((S, 1), lambda i: (0, 0)),
        ],
        out_specs=pl.BlockSpec((S, D), lambda i: (0, 0)),
        out_shape=jax.ShapeDtypeStruct((S, D), jnp.float32),
    )(pre_act, W_emb, thr)


TSB = 512


def _tsel_body(m_ref, t_ref):
    mv = m_ref[...]
    cur = jnp.concatenate(
        [mv[:, j * PADW:j * PADW + EBLK // CW] for j in range(NEB)], axis=-1)
    for _ in range(K - 1):
        mx = jnp.max(cur, axis=-1, keepdims=True)
        cur = jnp.where(cur == mx, _MSB, cur)
    t_ref[...] = jnp.max(cur, axis=-1, keepdims=True)


def _tsel(cmax_keys):
    return pl.pallas_call(
        _tsel_body,
        grid=(S // TSB,),
        in_specs=[pl.BlockSpec((TSB, NEB * PADW), lambda i: (i, 0))],
        out_specs=pl.BlockSpec((TSB, 1), lambda i: (i, 0)),
        out_shape=jax.ShapeDtypeStruct((S, 1), jnp.int32),
    )(cmax_keys)


def _iota16():
    return lax.iota(jnp.int32, L)


def _digit(skey, lvl):
    d = lax.shift_right_logical(skey, jnp.int32(28 - 4 * lvl)) & jnp.int32(15)
    if lvl == 0:
        d = d ^ jnp.int32(8)
    return d


def _pick_bin(hist, rank, lvl, thresh):
    htot = hist[pl.ds(0, L)]
    for j in range(1, 16):
        htot = htot + hist[pl.ds(j * L, L)]
    scum = plsc.cumsum(lax.rev(htot, (0,)))
    k = jnp.max(plsc.all_reduce_ffs(scum >= rank))
    b = 15 - k
    prev = jnp.sum(jnp.where(_iota16() == (k - 1), scum, 0))
    rank = rank - prev
    bvec = jnp.broadcast_to(b, (L,)).astype(jnp.int32)
    bkey = bvec ^ jnp.int32(8) if lvl == 0 else bvec
    thresh = thresh | lax.shift_left(bkey, jnp.int32(28 - 4 * lvl))
    return rank, bvec, thresh


def _zero_hist(hist):
    for j in range(16):
        hist[pl.ds(j * L, L)] = jnp.zeros((L,), jnp.int32)


def _one_level(src, nv0, n, rank, thresh, hist, buf, lvl, compact):
    _zero_hist(hist)
    if lvl == 0:
        nv = nv0
    else:
        nv = lax.div(n + 15, jnp.int32(16))

    def hist_body(i, carry):
        u = src(i)
        digit = _digit(u, lvl)
        if lvl == 0:
            mask = None
        else:
            mask = (_iota16() + i * L) < n
        plsc.addupdate_scatter(hist, [_iota16() * 16 + digit],
                               jnp.ones((L,), jnp.int32), mask=mask)
        return carry

    lax.fori_loop(0, nv, hist_body, jnp.int32(0))
    rank, bvec, thresh = _pick_bin(hist, rank, lvl, thresh)
    if not compact:
        return n, rank, thresh

    def comp_body(i, base):
        u = src(i)
        digit = _digit(u, lvl)
        keep = digit == bvec
        if lvl != 0:
            keep = keep & ((_iota16() + i * L) < n)
        pos = base + plsc.cumsum(keep.astype(jnp.int32)) - 1
        plsc.store_scatter(buf, [pos], u, mask=keep)
        return base + plsc.all_reduce_population_count(keep)

    base = lax.fori_loop(0, nv, comp_body, jnp.zeros((L,), jnp.int32))
    return jnp.max(base), rank, thresh


def _rest_levels(n, rank, thresh, hist, buf, start, nlevels):
    def buf_load(i):
        return buf[pl.ds(i * L, L)]

    for lvl in range(start, nlevels):
        compact = lvl < nlevels - 1

        def do(args, lvl=lvl, compact=compact):
            nn, rr, tt = args
            return _one_level(buf_load, None, nn, rr, tt, hist, buf,
                              lvl, compact)

        n, rank, thresh = lax.cond(n > L, do, lambda a: a, (n, rank, thresh))

    def fin(args):
        nn, rr, tt = args
        u = buf_load(0)
        valid = _iota16() < nn
        srt = plsc.sort_key_val(u, u, mask=valid, descending=True)
        tt = jnp.broadcast_to(
            jnp.sum(jnp.where(_iota16() == rr - 1, srt[1], 0)), (L,)
        ).astype(jnp.int32)
        return nn, rr, tt

    n, rank, thresh = lax.cond(n <= L, fin, lambda a: a, (n, rank, thresh))
    return thresh


def _radix_select(load_fn, nv0, n0, rank0, hist, buf, nlevels):
    n, rank, thresh = _one_level(load_fn, nv0, jnp.int32(n0),
                                 jnp.int32(rank0), jnp.zeros((L,), jnp.int32),
                                 hist, buf, 0, True)
    return _rest_levels(n, rank, thresh, hist, buf, 1, nlevels)


def _cand_select(cand_v, m, t_u, rank0, hist, buf):
    def comp_body(c, base):
        for k in range(CV):
            u = cand_v[c, pl.ds(k * L, L)]
            keep = u >= t_u
            pos = base + plsc.cumsum(keep.astype(jnp.int32)) - 1
            plsc.store_scatter(buf, [pos], u, mask=keep)
            base = base + plsc.all_reduce_population_count(keep)
        return base

    base = lax.fori_loop(0, m, comp_body, jnp.zeros((L,), jnp.int32))
    n1 = jnp.max(base)
    plsc.store_scatter(buf, [n1 + _iota16()],
                       jnp.full((L,), _MSB, jnp.int32), mask=None)

    def buf_load(i):
        return buf[pl.ds(i * L, L)]

    nv = lax.div(n1 + 15, jnp.int32(16))
    n, rank, thresh = _one_level(buf_load, nv, nv * L, jnp.int32(rank0),
                                 jnp.zeros((L,), jnp.int32), hist, buf,
                                 0, True)
    return _rest_levels(n, rank, thresh, hist, buf, 1, 8)


def _sc_body(cmax_hbm, pre2d_hbm, tsel_hbm, out_hbm, cm_a, cm_b, hist_v,
             sel_a, sel_b, cand_a, cand_b, ping_v, tv_v, out_v,
             semc_a, semc_b, semg_a, semg_b):
    cid = lax.axis_index("c")
    sid = lax.axis_index("s")
    wid = sid * NC + cid
    row0 = wid * RPW
    pltpu.sync_copy(tsel_hbm.at[pl.ds(row0, RPW)], tv_v)

    def start_cmax(r, cm_v, semc):
        pltpu.async_copy(cmax_hbm.at[row0 + r], cm_v, semc)

    def stage_ab(r, cm_v, sel_v, cand_v, semc, semg):
        pltpu.make_async_copy(cmax_hbm.at[0], cm_v, semc).wait()

        tvreg = tv_v[pl.ds(lax.mul(lax.shift_right_logical(r, 4), L), L)]
        t_u = tvreg.at[jnp.broadcast_to(r & (L - 1), (L,))].get(
            mode="promise_in_bounds")

        rowbase = (row0 + r) * NCHUNK
        fill = jnp.broadcast_to(rowbase, (L,)).astype(jnp.int32)
        for j in range(NCHUNK // L):
            sel_v[pl.ds(j * L, L)] = fill

        low8 = _iota16() < 8
        base = jnp.zeros((L,), jnp.int32)
        for jb in range(NEB):
            v = cm_v[pl.ds(jb * PADW, L)]
            keep = (v >= t_u) & low8
            pos = base + plsc.cumsum(keep.astype(jnp.int32)) - 1
            ids = rowbase + jb * 8 + _iota16()
            plsc.store_scatter(sel_v, [pos], ids, mask=keep)
            base = base + plsc.all_reduce_population_count(keep)

        m = jnp.max(base)
        g = lax.div(m + (GB - 1), jnp.int32(GB))

        def gat_body(j, carry):
            pltpu.async_copy(
                pre2d_hbm.at[sel_v.at[pl.ds(j * GB, GB)]],
                cand_v.at[pl.ds(j * GB, GB)], semg)
            return carry

        lax.fori_loop(0, g, gat_body, jnp.int32(0))
        return m, g, t_u

    def stage_c(r, cand_v, semg, m, g, t_u):
        def drain(j, carry):
            pltpu.make_async_copy(pre2d_hbm.at[pl.ds(0, GB)],
                                  cand_v.at[pl.ds(j * GB, GB)], semg).wait()
            return carry

        lax.fori_loop(0, g, drain, jnp.int32(0))

        def cand_load(i):
            chunk = lax.shift_right_logical(i, 3)
            off = (i & (CV - 1)) * L
            return cand_v[chunk, pl.ds(off, L)]

        v_u = _cand_select(cand_v, m, t_u, K, hist_v, ping_v)
        plsc.store_scatter(out_v, [jnp.broadcast_to(r, (L,)).astype(jnp.int32)],
                           v_u, mask=_iota16() == 0)

    bufs = ((cm_a, sel_a, cand_a, semc_a, semg_a),
            (cm_b, sel_b, cand_b, semc_b, semg_b))
    start_cmax(0, cm_a, semc_a)

    def pair_body(q, gs):
        mg = list(gs)
        for par in (0, 1):
            cm_v, sel_v, cand_v, semc, semg = bufs[par]
            cmo_v, _, _, semco, _ = bufs[1 - par]
            r = q * 2 + par

            def run_ab(_):
                def pf(c):
                    start_cmax(r + 1, cmo_v, semco)
                    return c

                lax.cond(r + 1 < RPW, pf, lambda c: c, jnp.int32(0))
                return stage_ab(r, cm_v, sel_v, cand_v, semc, semg)

            mg[par] = lax.cond(r < RPW, run_ab,
                               lambda _: (jnp.int32(0), jnp.int32(0),
                                          jnp.zeros((L,), jnp.int32)),
                               jnp.int32(0))

            _, cand_o, semg_o, mg_o = (bufs[1 - par][2], bufs[1 - par][2],
                                       bufs[1 - par][4], mg[1 - par])

            def run_c(c):
                stage_c(r - 1, cand_o, semg_o, mg_o[0], mg_o[1], mg_o[2])
                return c

            lax.cond((r >= 1) & (r <= RPW), run_c, lambda c: c, jnp.int32(0))
        return tuple(mg)

    zz = (jnp.int32(0), jnp.int32(0), jnp.zeros((L,), jnp.int32))
    lax.fori_loop(0, RPW // 2 + 1, pair_body, (zz, zz))
    pltpu.sync_copy(out_v, out_hbm.at[pl.ds(row0, RPW)])


@functools.partial(
    pl.kernel,
    out_type=jax.ShapeDtypeStruct((S,), jnp.int32),
    mesh=plsc.VectorSubcoreMesh(core_axis_name="c", subcore_axis_name="s"),
    compiler_params=pltpu.CompilerParams(needs_layout_passes=False),
    scratch_types=[
        pltpu.VMEM((NEB * PADW,), jnp.int32),
        pltpu.VMEM((NEB * PADW,), jnp.int32),
        pltpu.VMEM((256,), jnp.int32),
        pltpu.VMEM((NCHUNK,), jnp.int32),
        pltpu.VMEM((NCHUNK,), jnp.int32),
        pltpu.VMEM((NCHUNK, CW), jnp.int32),
        pltpu.VMEM((NCHUNK, CW), jnp.int32),
        pltpu.VMEM((C + L,), jnp.int32),
        pltpu.VMEM((RPW,), jnp.int32),
        pltpu.VMEM((RPW,), jnp.int32),
        pltpu.SemaphoreType.DMA,
        pltpu.SemaphoreType.DMA,
        pltpu.SemaphoreType.DMA,
        pltpu.SemaphoreType.DMA,
    ],
)
def _sc_thresh(cmax_hbm, pre2d_hbm, tsel_hbm, out_hbm, cm_a, cm_b, hist_v,
               sel_a, sel_b, cand_a, cand_b, ping_v, tv_v, out_v,
               semc_a, semc_b, semg_a, semg_b):
    _sc_body(cmax_hbm, pre2d_hbm, tsel_hbm, out_hbm, cm_a, cm_b, hist_v,
             sel_a, sel_b, cand_a, cand_b, ping_v, tv_v, out_v,
             semc_a, semc_b, semg_a, semg_b)


def kernel(activations, W_enc, b_enc, W_emb):
    B = activations.shape[0]
    act2d = activations.reshape(B * S, D)
    b2d = jnp.broadcast_to(b_enc.reshape(1, C), (8, C))
    pre_keys, cmax_keys = _encode(act2d, W_enc, b2d)
    tsel = _tsel(cmax_keys)
    thr_keys = _sc_thresh(cmax_keys, pre_keys.reshape(S * NCHUNK, CW),
                          tsel.reshape(S))
    out = _decode(pre_keys, W_emb, thr_keys.reshape(S, 1))
    return out.reshape(B, S, D)


# speedup vs baseline: 1.9930x; 1.0190x over previous
import functools

import jax
import jax.numpy as jnp
from jax import lax
from jax.experimental import pallas as pl
from jax.experimental.pallas import tpu as pltpu
from jax.experimental.pallas import tpu_sc as plsc

S, D, C, K = 2048, 768, 24576, 32
EBLK = 1024
NEB = C // EBLK
DBLK = 1024
NDB = C // DBLK
CW = 128
NCHUNK = C // CW
CV = CW // 16
PADW = 128
GB = 48
NC, NS, L = 2, 16, 16
NW = NC * NS
RPW = S // NW


import numpy as np

_MSB = np.int32(-2147483648)


def _encode_body(a_ref, w_ref, b_ref, o_ref, m_ref):
    a = a_ref[...]
    w = w_ref[...]
    acc = lax.dot_general(a, w, (((1,), (1,)), ((), ())),
                          preferred_element_type=jnp.float32)
    acc = acc + b_ref[0:1, :]
    xi = lax.bitcast_convert_type(acc, jnp.int32)
    o_ref[...] = jnp.where(xi >= 0, xi, ~xi ^ _MSB)
    cm = jnp.concatenate(
        [jnp.max(acc[:, j * CW:(j + 1) * CW], axis=-1, keepdims=True)
         for j in range(EBLK // CW)], axis=-1)
    ci = lax.bitcast_convert_type(cm, jnp.int32)
    cmk = jnp.where(ci >= 0, ci, ~ci ^ _MSB)
    pad = jnp.full((S, PADW - EBLK // CW), _MSB, jnp.int32)
    m_ref[...] = jnp.concatenate([cmk, pad], axis=-1)


def _encode(act2d, W_enc, b_enc2d):
    return pl.pallas_call(
        _encode_body,
        grid=(NEB,),
        in_specs=[
            pl.BlockSpec((S, D), lambda i: (0, 0)),
            pl.BlockSpec((EBLK, D), lambda i: (i, 0)),
            pl.BlockSpec((8, EBLK), lambda i: (0, i)),
        ],
        out_specs=[
            pl.BlockSpec((S, EBLK), lambda i: (0, i)),
            pl.BlockSpec((S, PADW), lambda i: (0, i)),
        ],
        out_shape=[
            jax.ShapeDtypeStruct((S, C), jnp.int32),
            jax.ShapeDtypeStruct((S, NEB * PADW), jnp.int32),
        ],
    )(act2d, W_enc, b_enc2d)


def _decode_body(p_ref, w_ref, t_ref, o_ref):
    i = pl.program_id(0)
    skey = p_ref[...]
    t = t_ref[...]
    xi = jnp.where(skey >= 0, skey, ~(skey ^ _MSB))
    p = lax.bitcast_convert_type(xi, jnp.float32)
    masked = jnp.where(skey >= t, p, 0.0)
    acc = lax.dot_general(masked, w_ref[...], (((1,), (1,)), ((), ())),
                          preferred_element_type=jnp.float32)

    @pl.when(i == 0)
    def _():
        o_ref[...] = acc

    @pl.when(i != 0)
    def _():
        o_ref[...] += acc


def _decode(pre_act, W_emb, thr):
    return pl.pallas_call(
        _decode_body,
        grid=(NDB,),
        in_specs=[
            pl.BlockSpec((S, DBLK), lambda i: (0, i)),
            pl.BlockSpec((D, DBLK), lambda i: (0, i)),
            pl.BlockSpec((S, 1), lambda i: (0, 0)),
        ],
        out_specs=pl.BlockSpec((S, D), lambda i: (0, 0)),
        out_shape=jax.ShapeDtypeStruct((S, D), jnp.float32),
    )(pre_act, W_emb, thr)


TSB = 512


def _tsel_body(m_ref, t_ref):
    mv = m_ref[...]
    cur = jnp.concatenate(
        [mv[:, j * PADW:j * PADW + EBLK // CW] for j in range(NEB)], axis=-1)
    for _ in range(K - 1):
        mx = jnp.max(cur, axis=-1, keepdims=True)
        cur = jnp.where(cur == mx, _MSB, cur)
    t_ref[...] = jnp.max(cur, axis=-1, keepdims=True)


def _tsel(cmax_keys):
    return pl.pallas_call(
        _tsel_body,
        grid=(S // TSB,),
        in_specs=[pl.BlockSpec((TSB, NEB * PADW), lambda i: (i, 0))],
        out_specs=pl.BlockSpec((TSB, 1), lambda i: (i, 0)),
        out_shape=jax.ShapeDtypeStruct((S, 1), jnp.int32),
    )(cmax_keys)


def _iota16():
    return lax.iota(jnp.int32, L)


def _digit(skey, lvl):
    d = lax.shift_right_logical(skey, jnp.int32(28 - 4 * lvl)) & jnp.int32(15)
    if lvl == 0:
        d = d ^ jnp.int32(8)
    return d


def _pick_bin(hist, rank, lvl, thresh):
    htot = hist[pl.ds(0, L)]
    for j in range(1, 16):
        htot = htot + hist[pl.ds(j * L, L)]
    scum = plsc.cumsum(lax.rev(htot, (0,)))
    k = jnp.max(plsc.all_reduce_ffs(scum >= rank))
    b = 15 - k
    prev = jnp.sum(jnp.where(_iota16() == (k - 1), scum, 0))
    rank = rank - prev
    bvec = jnp.broadcast_to(b, (L,)).astype(jnp.int32)
    bkey = bvec ^ jnp.int32(8) if lvl == 0 else bvec
    thresh = thresh | lax.shift_left(bkey, jnp.int32(28 - 4 * lvl))
    return rank, bvec, thresh


def _zero_hist(hist):
    for j in range(16):
        hist[pl.ds(j * L, L)] = jnp.zeros((L,), jnp.int32)


def _one_level(src, nv0, n, rank, thresh, hist, buf, lvl, compact):
    _zero_hist(hist)
    if lvl == 0:
        nv = nv0
    else:
        nv = lax.div(n + 15, jnp.int32(16))

    def hist_body(i, carry):
        u = src(i)
        digit = _digit(u, lvl)
        if lvl == 0:
            mask = None
        else:
            mask = (_iota16() + i * L) < n
        plsc.addupdate_scatter(hist, [_iota16() * 16 + digit],
                               jnp.ones((L,), jnp.int32), mask=mask)
        return carry

    lax.fori_loop(0, nv, hist_body, jnp.int32(0))
    rank, bvec, thresh = _pick_bin(hist, rank, lvl, thresh)
    if not compact:
        return n, rank, thresh

    def comp_body(i, base):
        u = src(i)
        digit = _digit(u, lvl)
        keep = digit == bvec
        if lvl != 0:
            keep = keep & ((_iota16() + i * L) < n)
        pos = base + plsc.cumsum(keep.astype(jnp.int32)) - 1
        plsc.store_scatter(buf, [pos], u, mask=keep)
        return base + plsc.all_reduce_population_count(keep)

    base = lax.fori_loop(0, nv, comp_body, jnp.zeros((L,), jnp.int32))
    return jnp.max(base), rank, thresh


def _rest_levels(n, rank, thresh, hist, buf, start, nlevels):
    def buf_load(i):
        return buf[pl.ds(i * L, L)]

    for lvl in range(start, nlevels):
        compact = lvl < nlevels - 1

        def do(args, lvl=lvl, compact=compact):
            nn, rr, tt = args
            return _one_level(buf_load, None, nn, rr, tt, hist, buf,
                              lvl, compact)

        n, rank, thresh = lax.cond(n > L, do, lambda a: a, (n, rank, thresh))

    def fin(args):
        nn, rr, tt = args
        u = buf_load(0)
        valid = _iota16() < nn
        srt = plsc.sort_key_val(u, u, mask=valid, descending=True)
        tt = jnp.broadcast_to(
            jnp.sum(jnp.where(_iota16() == rr - 1, srt[1], 0)), (L,)
        ).astype(jnp.int32)
        return nn, rr, tt

    n, rank, thresh = lax.cond(n <= L, fin, lambda a: a, (n, rank, thresh))
    return thresh


def _radix_select(load_fn, nv0, n0, rank0, hist, buf, nlevels):
    n, rank, thresh = _one_level(load_fn, nv0, jnp.int32(n0),
                                 jnp.int32(rank0), jnp.zeros((L,), jnp.int32),
                                 hist, buf, 0, True)
    return _rest_levels(n, rank, thresh, hist, buf, 1, nlevels)


def _cand_select(cand_v, m, t_u, rank0, hist, buf):
    def comp_body(c, base):
        for k in range(CV):
            u = cand_v[c, pl.ds(k * L, L)]
            keep = u >= t_u
            pos = base + plsc.cumsum(keep.astype(jnp.int32)) - 1
            plsc.store_scatter(buf, [pos], u, mask=keep)
            base = base + plsc.all_reduce_population_count(keep)
        return base

    base = lax.fori_loop(0, m, comp_body, jnp.zeros((L,), jnp.int32))
    n1 = jnp.max(base)
    plsc.store_scatter(buf, [n1 + _iota16()],
                       jnp.full((L,), _MSB, jnp.int32), mask=None)

    def buf_load(i):
        return buf[pl.ds(i * L, L)]

    nv = lax.div(n1 + 15, jnp.int32(16))
    n, rank, thresh = _one_level(buf_load, nv, nv * L, jnp.int32(rank0),
                                 jnp.zeros((L,), jnp.int32), hist, buf,
                                 0, True)
    return _rest_levels(n, rank, thresh, hist, buf, 1, 8)


def _sc_body(cmax_hbm, pre2d_hbm, tsel_hbm, out_hbm, cm_a, cm_b, hist_v,
             sel_a, sel_b, cand_a, cand_b, ping_v, tv_v, out_v,
             semc_a, semc_b, semg_a, semg_b):
    cid = lax.axis_index("c")
    sid = lax.axis_index("s")
    wid = sid * NC + cid
    row0 = wid * RPW
    pltpu.sync_copy(tsel_hbm.at[pl.ds(row0, RPW)], tv_v)

    def start_cmax(r, cm_v, semc):
        pltpu.async_copy(cmax_hbm.at[row0 + r], cm_v, semc)

    def stage_ab(r, cm_v, sel_v, cand_v, semc, semg):
        pltpu.make_async_copy(cmax_hbm.at[0], cm_v, semc).wait()

        tvreg = tv_v[pl.ds(lax.mul(lax.shift_right_logical(r, 4), L), L)]
        t_u = tvreg.at[jnp.broadcast_to(r & (L - 1), (L,))].get(
            mode="promise_in_bounds")

        rowbase = (row0 + r) * NCHUNK
        fill = jnp.broadcast_to(rowbase, (L,)).astype(jnp.int32)
        for j in range(NCHUNK // L):
            sel_v[pl.ds(j * L, L)] = fill

        low8 = _iota16() < 8
        base = jnp.zeros((L,), jnp.int32)
        for jb in range(NEB):
            v = cm_v[pl.ds(jb * PADW, L)]
            keep = (v >= t_u) & low8
            pos = base + plsc.cumsum(keep.astype(jnp.int32)) - 1
            ids = rowbase + jb * 8 + _iota16()
            plsc.store_scatter(sel_v, [pos], ids, mask=keep)
            base = base + plsc.all_reduce_population_count(keep)

        m = jnp.max(base)
        g = lax.div(m + (GB - 1), jnp.int32(GB))

        def gat_body(j, carry):
            pltpu.async_copy(
                pre2d_hbm.at[sel_v.at[pl.ds(j * GB, GB)]],
                cand_v.at[pl.ds(j * GB, GB)], semg)
            return carry

        lax.fori_loop(0, g, gat_body, jnp.int32(0))
        return m, g, t_u

    def stage_c(r, cand_v, semg, m, g, t_u):
        def drain(j, carry):
            pltpu.make_async_copy(pre2d_hbm.at[pl.ds(0, GB)],
                                  cand_v.at[pl.ds(j * GB, GB)], semg).wait()
            return carry

        lax.fori_loop(0, g, drain, jnp.int32(0))

        def cand_load(i):
            chunk = lax.shift_right_logical(i, 3)
            off = (i & (CV - 1)) * L
            return cand_v[chunk, pl.ds(off, L)]

        v_u = _cand_select(cand_v, m, t_u, K, hist_v, ping_v)
        plsc.store_scatter(out_v, [jnp.broadcast_to(r, (L,)).astype(jnp.int32)],
                           v_u, mask=_iota16() == 0)

    bufs = ((cm_a, sel_a, cand_a, semc_a, semg_a),
            (cm_b, sel_b, cand_b, semc_b, semg_b))
    start_cmax(0, cm_a, semc_a)

    def pair_body(q, gs):
        mg = list(gs)
        for par in (0, 1):
            cm_v, sel_v, cand_v, semc, semg = bufs[par]
            cmo_v, _, _, semco, _ = bufs[1 - par]
            r = q * 2 + par

            def run_ab(_):
                def pf(c):
                    start_cmax(r + 1, cmo_v, semco)
                    return c

                lax.cond(r + 1 < RPW, pf, lambda c: c, jnp.int32(0))
                return stage_ab(r, cm_v, sel_v, cand_v, semc, semg)

            mg[par] = lax.cond(r < RPW, run_ab,
                               lambda _: (jnp.int32(0), jnp.int32(0),
                                          jnp.zeros((L,), jnp.int32)),
                               jnp.int32(0))

            _, cand_o, semg_o, mg_o = (bufs[1 - par][2], bufs[1 - par][2],
                                       bufs[1 - par][4], mg[1 - par])

            def run_c(c):
                stage_c(r - 1, cand_o, semg_o, mg_o[0], mg_o[1], mg_o[2])
                return c

            lax.cond((r >= 1) & (r <= RPW), run_c, lambda c: c, jnp.int32(0))
        return tuple(mg)

    zz = (jnp.int32(0), jnp.int32(0), jnp.zeros((L,), jnp.int32))
    lax.fori_loop(0, RPW // 2 + 1, pair_body, (zz, zz))
    pltpu.sync_copy(out_v, out_hbm.at[pl.ds(row0, RPW)])


@functools.partial(
    pl.kernel,
    out_type=jax.ShapeDtypeStruct((S,), jnp.int32),
    mesh=plsc.VectorSubcoreMesh(core_axis_name="c", subcore_axis_name="s"),
    compiler_params=pltpu.CompilerParams(needs_layout_passes=False),
    scratch_types=[
        pltpu.VMEM((NEB * PADW,), jnp.int32),
        pltpu.VMEM((NEB * PADW,), jnp.int32),
        pltpu.VMEM((256,), jnp.int32),
        pltpu.VMEM((NCHUNK,), jnp.int32),
        pltpu.VMEM((NCHUNK,), jnp.int32),
        pltpu.VMEM((NCHUNK, CW), jnp.int32),
        pltpu.VMEM((NCHUNK, CW), jnp.int32),
        pltpu.VMEM((C + L,), jnp.int32),
        pltpu.VMEM((RPW,), jnp.int32),
        pltpu.VMEM((RPW,), jnp.int32),
        pltpu.SemaphoreType.DMA,
        pltpu.SemaphoreType.DMA,
        pltpu.SemaphoreType.DMA,
        pltpu.SemaphoreType.DMA,
    ],
)
def _sc_thresh(cmax_hbm, pre2d_hbm, tsel_hbm, out_hbm, cm_a, cm_b, hist_v,
               sel_a, sel_b, cand_a, cand_b, ping_v, tv_v, out_v,
               semc_a, semc_b, semg_a, semg_b):
    _sc_body(cmax_hbm, pre2d_hbm, tsel_hbm, out_hbm, cm_a, cm_b, hist_v,
             sel_a, sel_b, cand_a, cand_b, ping_v, tv_v, out_v,
             semc_a, semc_b, semg_a, semg_b)


def kernel(activations, W_enc, b_enc, W_emb):
    B = activations.shape[0]
    act2d = activations.reshape(B * S, D)
    b2d = jnp.broadcast_to(b_enc.reshape(1, C), (8, C))
    pre_keys, cmax_keys = _encode(act2d, W_enc, b2d)
    tsel = _tsel(cmax_keys)
    thr_keys = _sc_thresh(cmax_keys, pre_keys.reshape(S * NCHUNK, CW),
                          tsel.reshape(S))
    out = _decode(pre_keys, W_emb, thr_keys.reshape(S, 1))
    return out.reshape(B, S, D)
